# baseline (speedup 1.0000x reference)
import jax
import jax.numpy as jnp
from jax.experimental import pallas as pl
from jax.experimental.pallas import tpu as pltpu

_CELLS = 2
_IPS = 8


def _encoder_body(x_ref, bn_scale_ref, bn_shift_ref, w0_ref, b0_ref,
                  w1_ref, b1_ref, o_ref, *scratch):
    H = x_ref.shape[1]
    W = x_ref.shape[2]
    C = x_ref.shape[3]
    HW = H * W
    stgs = scratch
    rpc = H // 2

    for stg in stgs:
        stg[...] = jnp.zeros(stg.shape, stg.dtype)

    def bn_conv(stg, x2d, bn_row, w_ref, c, b):
        scale = bn_scale_ref[bn_row:bn_row + 1, :]
        shift = bn_shift_ref[bn_row:bn_row + 1, :]
        bnx = (x2d * scale + shift).astype(jnp.bfloat16).reshape(H, W, C)
        stg[1:H + 1, :, C:2 * C] = bnx
        stg[1:H + 1, 1:W, 0:C] = bnx[:, :W - 1, :]
        stg[1:H + 1, 0:W - 1, 2 * C:3 * C] = bnx[:, 1:, :]
        outs = []
        for j in range(2):
            acc = b
            for kh in range(3):
                lhs = stg[kh + rpc * j:kh + rpc * j + rpc, :, :]
                wk = w_ref[c, kh * 3 * C:(kh + 1) * 3 * C, :]
                acc = acc + jnp.dot(lhs.reshape(rpc * W, 3 * C), wk,
                                    preferred_element_type=jnp.float32)
            outs.append(acc)
        return jnp.concatenate(outs, axis=0)

    cells = [x_ref[i].reshape(HW, C).astype(jnp.float32) for i in range(_IPS)]
    for c in range(_CELLS):
        y0 = [bn_conv(stgs[i], cells[i], 2 * c + 0,
                      w0_ref, c, b0_ref[c]) for i in range(_IPS)]
        n1 = [jnp.maximum(y0[i][:, :C], 0.0) for i in range(_IPS)]
        y1 = [bn_conv(stgs[i], n1[i], 2 * c + 1,
                      w1_ref, c, b1_ref[c]) for i in range(_IPS)]
        cells = [y0[i][:, C:] + jnp.maximum(y1[i], 0.0) for i in range(_IPS)]
    for i in range(_IPS):
        o_ref[i] = jnp.sum(cells[i], axis=0, keepdims=True) * (1.0 / HW)


def _head_body(p_ref, hw_ref, hb_ref, o_ref):
    o_ref[...] = jnp.dot(p_ref[...], hw_ref[...],
                         preferred_element_type=jnp.float32) + hb_ref[...]


def kernel(x, bn_scale, bn_shift, w0, b0, w1, b1, head_w, head_b):
    x = jnp.transpose(x, (0, 2, 3, 1)).astype(jnp.bfloat16)
    B, H, W, C = x.shape
    K = head_w.shape[1]
    nine_c = 9 * C

    pooled = pl.pallas_call(
        _encoder_body,
        out_shape=jax.ShapeDtypeStruct((B, 1, C), jnp.float32),
        grid=(B // _IPS,),
        in_specs=[
            pl.BlockSpec((_IPS, H, W, C), lambda b: (b, 0, 0, 0)),
            pl.BlockSpec((2 * _CELLS, C), lambda b: (0, 0)),
            pl.BlockSpec((2 * _CELLS, C), lambda b: (0, 0)),
            pl.BlockSpec((_CELLS, nine_c, 2 * C), lambda b: (0, 0, 0)),
            pl.BlockSpec((_CELLS, 1, 2 * C), lambda b: (0, 0, 0)),
            pl.BlockSpec((_CELLS, nine_c, C), lambda b: (0, 0, 0)),
            pl.BlockSpec((_CELLS, 1, C), lambda b: (0, 0, 0)),
        ],
        out_specs=pl.BlockSpec((_IPS, 1, C), lambda b: (b, 0, 0)),
        scratch_shapes=[pltpu.VMEM((H + 2, W, 3 * C), jnp.bfloat16)
                        for _ in range(_IPS)],
        compiler_params=pltpu.CompilerParams(dimension_semantics=("parallel",)),
    )(x, bn_scale, bn_shift, w0.astype(jnp.bfloat16), b0,
      w1.astype(jnp.bfloat16), b1)

    logits = pl.pallas_call(
        _head_body,
        out_shape=jax.ShapeDtypeStruct((B, K), jnp.float32),
    )(pooled.reshape(B, C), head_w, head_b)
    return logits


# speedup vs baseline: 1.0639x; 1.0639x over previous
import jax
import jax.numpy as jnp
from jax.experimental import pallas as pl
from jax.experimental.pallas import tpu as pltpu

_CELLS = 2
_IPS = 4


def _encoder_body(x_ref, bn_scale_ref, bn_shift_ref, w0_ref, b0_ref,
                  w1_ref, b1_ref, o_ref, *scratch):
    H = x_ref.shape[1]
    W = x_ref.shape[2]
    C = x_ref.shape[3]
    HW = H * W
    stgs = scratch
    rpc = H // 4

    for stg in stgs:
        stg[...] = jnp.zeros(stg.shape, stg.dtype)

    def bn_conv(stg, x2d, bn_row, w_ref, c, b):
        scale = bn_scale_ref[bn_row:bn_row + 1, :]
        shift = bn_shift_ref[bn_row:bn_row + 1, :]
        bnx = (x2d * scale + shift).astype(jnp.bfloat16).reshape(H, W, C)
        stg[1:H + 1, :, C:2 * C] = bnx
        stg[1:H + 1, 1:W, 0:C] = bnx[:, :W - 1, :]
        stg[1:H + 1, 0:W - 1, 2 * C:3 * C] = bnx[:, 1:, :]
        outs = []
        for j in range(4):
            acc = b
            for kh in range(3):
                lhs = stg[kh + rpc * j:kh + rpc * j + rpc, :, :]
                wk = w_ref[c, kh * 3 * C:(kh + 1) * 3 * C, :]
                acc = acc + jnp.dot(lhs.reshape(rpc * W, 3 * C), wk,
                                    preferred_element_type=jnp.float32)
            outs.append(acc)
        return jnp.concatenate(outs, axis=0)

    cells = [x_ref[i].reshape(HW, C).astype(jnp.float32) for i in range(_IPS)]
    for c in range(_CELLS):
        y0 = [bn_conv(stgs[i], cells[i], 2 * c + 0,
                      w0_ref, c, b0_ref[c]) for i in range(_IPS)]
        n1 = [jnp.maximum(y0[i][:, :C], 0.0) for i in range(_IPS)]
        y1 = [bn_conv(stgs[i], n1[i], 2 * c + 1,
                      w1_ref, c, b1_ref[c]) for i in range(_IPS)]
        cells = [y0[i][:, C:] + jnp.maximum(y1[i], 0.0) for i in range(_IPS)]
    for i in range(_IPS):
        o_ref[i] = jnp.sum(cells[i], axis=0, keepdims=True) * (1.0 / HW)


def _head_body(p_ref, hw_ref, hb_ref, o_ref):
    o_ref[...] = jnp.dot(p_ref[...], hw_ref[...],
                         preferred_element_type=jnp.float32) + hb_ref[...]


def kernel(x, bn_scale, bn_shift, w0, b0, w1, b1, head_w, head_b):
    x = jnp.transpose(x, (0, 2, 3, 1)).astype(jnp.bfloat16)
    B, H, W, C = x.shape
    K = head_w.shape[1]
    nine_c = 9 * C

    pooled = pl.pallas_call(
        _encoder_body,
        out_shape=jax.ShapeDtypeStruct((B, 1, C), jnp.float32),
        grid=(B // _IPS,),
        in_specs=[
            pl.BlockSpec((_IPS, H, W, C), lambda b: (b, 0, 0, 0)),
            pl.BlockSpec((2 * _CELLS, C), lambda b: (0, 0)),
            pl.BlockSpec((2 * _CELLS, C), lambda b: (0, 0)),
            pl.BlockSpec((_CELLS, nine_c, 2 * C), lambda b: (0, 0, 0)),
            pl.BlockSpec((_CELLS, 1, 2 * C), lambda b: (0, 0, 0)),
            pl.BlockSpec((_CELLS, nine_c, C), lambda b: (0, 0, 0)),
            pl.BlockSpec((_CELLS, 1, C), lambda b: (0, 0, 0)),
        ],
        out_specs=pl.BlockSpec((_IPS, 1, C), lambda b: (b, 0, 0)),
        scratch_shapes=[pltpu.VMEM((H + 2, W, 3 * C), jnp.bfloat16)
                        for _ in range(_IPS)],
        compiler_params=pltpu.CompilerParams(dimension_semantics=("parallel",)),
    )(x, bn_scale, bn_shift, w0.astype(jnp.bfloat16), b0,
      w1.astype(jnp.bfloat16), b1)

    logits = pl.pallas_call(
        _head_body,
        out_shape=jax.ShapeDtypeStruct((B, K), jnp.float32),
    )(pooled.reshape(B, C), head_w, head_b)
    return logits
